# baseline (speedup 1.0000x reference)
import functools

import jax
import jax.numpy as jnp
from jax import lax
from jax.experimental import pallas as pl
from jax.experimental.pallas import tpu as pltpu
from jax.experimental.pallas import tpu_sc as plsc

TOTAL_COARSE = 8192
TOTAL_FINE = 65536
C = 512

NUM_CORES = 2
NUM_SUBCORES = 16
NW = NUM_CORES * NUM_SUBCORES
B_PER_W = TOTAL_FINE // NW
CHUNK = 16
NCHUNK = B_PER_W // CHUNK
NBUF = 8

_mesh = plsc.VectorSubcoreMesh(
    core_axis_name="c", subcore_axis_name="s",
    num_cores=NUM_CORES, num_subcores=NUM_SUBCORES)


@functools.partial(
    pl.kernel,
    out_type=jax.ShapeDtypeStruct((TOTAL_FINE, C), jnp.float32),
    mesh=_mesh,
    scratch_types=(
        [pltpu.VMEM((B_PER_W,), jnp.int32)]
        + [pltpu.VMEM((CHUNK, C), jnp.float32) for _ in range(NBUF)]
        + [pltpu.SemaphoreType.DMA for _ in range(2 * NBUF)]
    ),
)
def _sc_gather(table_hbm, idx_hbm, out_hbm, idx_v, *rest):
    bufs = rest[:NBUF]
    gsems = rest[NBUF:2 * NBUF]
    ssems = rest[2 * NBUF:]
    wid = lax.axis_index("s") * NUM_CORES + lax.axis_index("c")
    base = pl.multiple_of(wid * B_PER_W, B_PER_W)
    pltpu.sync_copy(idx_hbm.at[pl.ds(base, B_PER_W)], idx_v)

    def gather(g, b):
        idx_slice = idx_v.at[pl.ds(g * CHUNK, CHUNK)]
        return pltpu.make_async_copy(table_hbm.at[idx_slice], bufs[b], gsems[b])

    def scatter(g, b):
        off = pl.multiple_of(base + g * CHUNK, CHUNK)
        return pltpu.make_async_copy(bufs[b], out_hbm.at[pl.ds(off, CHUNK)],
                                     ssems[b])

    for b in range(NBUF):
        gather(b, b).start()

    def body(g):
        for b in range(NBUF):
            gather(g + b, b).wait()
            scatter(g + b, b).start()
        for b in range(NBUF):
            scatter(g + b, b).wait()
            gather(g + b + NBUF, b).start()

    pl.loop(0, NCHUNK - NBUF, step=NBUF)(body)

    for b in range(NBUF):
        gather(NCHUNK - NBUF + b, b).wait()
        scatter(NCHUNK - NBUF + b, b).start()
    for b in range(NBUF):
        scatter(NCHUNK - NBUF + b, b).wait()


def kernel(coarse_data, fine_to_coarse):
    return _sc_gather(coarse_data, fine_to_coarse)
